# baseline (speedup 1.0000x reference)
import functools
import math

import jax
import jax.numpy as jnp
from jax import lax
from jax.experimental import pallas as pl
from jax.experimental.pallas import tpu as pltpu
from jax.experimental.pallas import tpu_sc as plsc

_SIZE = 32000
_N = 4096
_SMOOTHING = 0.1
_EPS = _SMOOTHING / (_SIZE - 2)
_CONF = 1.0 - _SMOOTHING
_C0 = _N * (_SIZE - 1) * _EPS * math.log(_EPS)
_DCONST = _CONF * math.log(_CONF) - _EPS * math.log(_EPS)


_BR = 512
_BC = 3200
_RI = _N // _BR
_CJ = _SIZE // _BC


def _tc_body(x_ref, out_ref, acc_ref):
    i = pl.program_id(0)
    j = pl.program_id(1)

    @pl.when((i == 0) & (j == 0))
    def _init():
        acc_ref[0] = 0.0
        acc_ref[1] = 0.0

    xt = x_ref[...]
    acc_ref[0] += jnp.sum(xt)

    @pl.when(j == 0)
    def _first_col_block():
        acc_ref[1] += jnp.sum(xt[:, 0:1])

    @pl.when((i == _RI - 1) & (j == _CJ - 1))
    def _finalize():
        out_ref[0, 0] = _C0 - _EPS * acc_ref[0] + _EPS * acc_ref[1]


def _tc_sums(x):
    return pl.pallas_call(
        _tc_body,
        grid=(_RI, _CJ),
        in_specs=[pl.BlockSpec((_BR, _BC), lambda i, j: (i, j))],
        out_specs=pl.BlockSpec(memory_space=pltpu.SMEM),
        out_shape=jax.ShapeDtypeStruct((1, 1), jnp.float32),
        scratch_shapes=[pltpu.SMEM((2,), jnp.float32)],
    )(x)


_NC = 2
_NS = 16
_NW = _NC * _NS
_PER_W = _N // _NW
_L = 16


def _sc_body(xflat_hbm, tgt_hbm, out_hbm, tgt_v, idx_v, vals_v, part_v, sem):
    wid = lax.axis_index("s") * _NC + lax.axis_index("c")
    base = wid * _PER_W
    pltpu.sync_copy(tgt_hbm.at[pl.ds(base, _PER_W)], tgt_v)
    for k in range(_PER_W // _L):
        t = tgt_v[pl.ds(k * _L, _L)]
        row = base + k * _L + lax.iota(jnp.int32, _L)
        idx_v[pl.ds(k * _L, _L)] = row * _SIZE + t
    pltpu.async_copy(xflat_hbm.at[idx_v], vals_v, sem).wait()
    acc = jnp.zeros((_L,), jnp.float32)
    cnt = jnp.zeros((_L,), jnp.float32)
    for k in range(_PER_W // _L):
        t = tgt_v[pl.ds(k * _L, _L)]
        v = vals_v[pl.ds(k * _L, _L)]
        m = t != 0
        acc = acc + jnp.where(m, v, 0.0)
        cnt = cnt + jnp.where(m, 1.0, 0.0)
    part_v[...] = _DCONST * cnt - (_CONF - _EPS) * acc
    pltpu.sync_copy(part_v, out_hbm.at[wid])


_sc_mesh = plsc.VectorSubcoreMesh(core_axis_name="c", subcore_axis_name="s")

_sc_gather = functools.partial(
    pl.kernel,
    mesh=_sc_mesh,
    out_type=jax.ShapeDtypeStruct((_NW, _L), jnp.float32),
    scratch_types=[
        pltpu.VMEM((_PER_W,), jnp.int32),
        pltpu.VMEM((_PER_W,), jnp.int32),
        pltpu.VMEM((_PER_W,), jnp.float32),
        pltpu.VMEM((_L,), jnp.float32),
        pltpu.SemaphoreType.DMA,
    ],
)(_sc_body)


def kernel(x, tgt):
    tc_out = _tc_sums(x)
    sc_out = _sc_gather(x.reshape(-1), tgt.astype(jnp.int32))
    return tc_out[0, 0] + jnp.sum(sc_out)


# speedup vs baseline: 3.1691x; 3.1691x over previous
import functools
import math

import jax
import jax.numpy as jnp
from jax import lax
from jax.experimental import pallas as pl
from jax.experimental.pallas import tpu as pltpu
from jax.experimental.pallas import tpu_sc as plsc

_SIZE = 32000
_N = 4096
_SMOOTHING = 0.1
_EPS = _SMOOTHING / (_SIZE - 2)
_CONF = 1.0 - _SMOOTHING
_C0 = _N * (_SIZE - 1) * _EPS * math.log(_EPS)
_DCONST = _CONF * math.log(_CONF) - _EPS * math.log(_EPS)


_BR = 64
_RI = _N // _BR


def _tc_body(x_ref, out_ref, acc_ref, c0_ref):
    i = pl.program_id(0)

    @pl.when(i == 0)
    def _init():
        acc_ref[...] = jnp.zeros_like(acc_ref)
        c0_ref[0] = 0.0

    xt = x_ref[...]
    partial = xt[0:8, :]
    for g in range(1, _BR // 8):
        partial = partial + xt[8 * g:8 * (g + 1), :]
    acc_ref[...] += partial
    c0_ref[0] += jnp.sum(xt[:, 0:1])

    @pl.when(i == _RI - 1)
    def _finalize():
        out_ref[0, 0] = (_C0
                         - _EPS * jnp.sum(acc_ref[...])
                         + _EPS * c0_ref[0])


def _tc_sums(x):
    return pl.pallas_call(
        _tc_body,
        grid=(_RI,),
        in_specs=[pl.BlockSpec((_BR, _SIZE), lambda i: (i, 0))],
        out_specs=pl.BlockSpec(memory_space=pltpu.SMEM),
        out_shape=jax.ShapeDtypeStruct((1, 1), jnp.float32),
        scratch_shapes=[
            pltpu.VMEM((8, _SIZE), jnp.float32),
            pltpu.SMEM((1,), jnp.float32),
        ],
    )(x)


_NC = 2
_NS = 16
_NW = _NC * _NS
_PER_W = _N // _NW
_L = 16


def _sc_body(x_hbm, tgt_hbm, out_hbm, tgt_v, chunk_v, part_v, sem):
    wid = lax.axis_index("s") * _NC + lax.axis_index("c")
    base = wid * _PER_W
    pltpu.sync_copy(tgt_hbm.at[pl.ds(base, _PER_W)], tgt_v)
    lane = lax.iota(jnp.int32, _L)
    acc = jnp.zeros((_L,), jnp.float32)
    cnt = jnp.zeros((_L,), jnp.float32)
    for k in range(_PER_W // _L):
        t = tgt_v[pl.ds(k * _L, _L)]
        t15 = jnp.bitwise_and(t, 15)
        m = t != 0
        cnt = cnt + jnp.where(m, 1.0, 0.0)
        scal = [t[r] for r in range(_L)]
        copies = []
        for r in range(_L):
            c128 = pl.multiple_of(jnp.bitwise_and(scal[r], jnp.int32(~127)), 128)
            row8 = pl.multiple_of(base + k * _L + 8 * (r // 8), 8)
            copies.append(pltpu.async_copy(
                x_hbm.at[pl.ds(row8, 8), pl.ds(c128, 128)],
                chunk_v.at[r], sem))
        for c in copies:
            c.wait()
        for r in range(_L):
            off16 = jnp.bitwise_and(scal[r], jnp.int32(112))
            c_r = chunk_v[r, r % 8, pl.ds(off16, _L)]
            g = c_r.at[t15].get(mode="promise_in_bounds")
            acc = acc + jnp.where((lane == r) & m, g, 0.0)
    part_v[...] = _DCONST * cnt - (_CONF - _EPS) * acc
    pltpu.sync_copy(part_v, out_hbm.at[wid])


_sc_mesh = plsc.VectorSubcoreMesh(core_axis_name="c", subcore_axis_name="s")

_sc_gather = functools.partial(
    pl.kernel,
    mesh=_sc_mesh,
    out_type=jax.ShapeDtypeStruct((_NW, _L), jnp.float32),
    scratch_types=[
        pltpu.VMEM((_PER_W,), jnp.int32),
        pltpu.VMEM((_L, 8, 128), jnp.float32),
        pltpu.VMEM((_L,), jnp.float32),
        pltpu.SemaphoreType.DMA,
    ],
)(_sc_body)


def kernel(x, tgt):
    tc_out = _tc_sums(x)
    sc_out = _sc_gather(x, tgt.astype(jnp.int32))
    return tc_out[0, 0] + jnp.sum(sc_out)
